# baseline (device time: 41646 ns/iter reference)
import jax
import jax.numpy as jnp
from jax import lax
from jax.experimental import pallas as pl
from jax.experimental.pallas import tpu as pltpu

N_DEV = 4


def kernel(x, Wp):
    b, h, w, c = x.shape
    cout = Wp.shape[1]
    hw = h * w
    n_global = (h * N_DEV) * w
    x3 = x.reshape(b, hw, c)

    def body(x_ref, wp_ref, out_ref, stats_ref, send_sems, recv_sems):
        my = lax.axis_index("i")

        for bi in range(b):
            xb = x_ref[bi]
            stats_ref[N_DEV - 1, 2 * bi : 2 * bi + 1, :] = jnp.sum(
                xb, axis=0, keepdims=True
            )
            stats_ref[N_DEV - 1, 2 * bi + 1 : 2 * bi + 2, :] = jnp.sum(
                xb * xb, axis=0, keepdims=True
            )

        sends = []
        for d in (1, 2, 3):
            rdma = pltpu.make_async_remote_copy(
                src_ref=stats_ref.at[N_DEV - 1],
                dst_ref=stats_ref.at[d - 1],
                send_sem=send_sems.at[d - 1],
                recv_sem=recv_sems.at[d - 1],
                device_id=((my + d) % N_DEV,),
                device_id_type=pl.DeviceIdType.MESH,
            )
            rdma.start()
            sends.append(rdma)

        for d in (1, 2, 3):
            recv = pltpu.make_async_remote_copy(
                src_ref=stats_ref.at[N_DEV - 1],
                dst_ref=stats_ref.at[d - 1],
                send_sem=send_sems.at[d - 1],
                recv_sem=recv_sems.at[d - 1],
                device_id=((my + d) % N_DEV,),
                device_id_type=pl.DeviceIdType.MESH,
            )
            recv.wait_recv()
        for rdma in sends:
            rdma.wait_send()

        eps = 1e-5
        inv_n = 1.0 / float(n_global)
        wp = wp_ref[:, :]
        for bi in range(b):
            ssum = (
                stats_ref[0, 2 * bi : 2 * bi + 1, :]
                + stats_ref[1, 2 * bi : 2 * bi + 1, :]
                + stats_ref[2, 2 * bi : 2 * bi + 1, :]
                + stats_ref[3, 2 * bi : 2 * bi + 1, :]
            )
            ssq = (
                stats_ref[0, 2 * bi + 1 : 2 * bi + 2, :]
                + stats_ref[1, 2 * bi + 1 : 2 * bi + 2, :]
                + stats_ref[2, 2 * bi + 1 : 2 * bi + 2, :]
                + stats_ref[3, 2 * bi + 1 : 2 * bi + 2, :]
            )
            mean = ssum * inv_n
            var = ssq * inv_n - mean * mean
            scale = lax.rsqrt(var + eps)
            xb = x_ref[bi]
            hh = (xb - mean) * scale
            a = hh * jax.nn.sigmoid(hh)
            out_ref[bi] = jnp.dot(a, wp, preferred_element_type=jnp.float32)

    out = pl.pallas_call(
        body,
        out_shape=jax.ShapeDtypeStruct((b, hw, cout), jnp.float32),
        in_specs=[
            pl.BlockSpec(memory_space=pltpu.VMEM),
            pl.BlockSpec(memory_space=pltpu.VMEM),
        ],
        out_specs=pl.BlockSpec(memory_space=pltpu.VMEM),
        scratch_shapes=[
            pltpu.VMEM((N_DEV, 2 * b, c), jnp.float32),
            pltpu.SemaphoreType.DMA((N_DEV - 1,)),
            pltpu.SemaphoreType.DMA((N_DEV - 1,)),
        ],
    )(x3, Wp)
    return out.reshape(b, h, w, cout)


# device time: 37179 ns/iter; 1.1201x vs baseline; 1.1201x over previous
import jax
import jax.numpy as jnp
from jax import lax
from jax.experimental import pallas as pl
from jax.experimental.pallas import tpu as pltpu

N_DEV = 4
NCHUNK = 4


def kernel(x, Wp):
    b, h, w, c = x.shape
    cout = Wp.shape[1]
    hw = h * w
    rows = hw // NCHUNK
    n_global = (h * N_DEV) * w
    x3 = x.reshape(b, hw, c)

    def body(
        x_hbm,
        wp_ref,
        out_hbm,
        x_vmem,
        outbuf,
        stats_ref,
        copy_sems,
        out_sems,
        send_sems,
        recv_sems,
    ):
        my = lax.axis_index("i")

        copies = []
        for bi in range(b):
            for ci in range(NCHUNK):
                cp = pltpu.make_async_copy(
                    x_hbm.at[bi, pl.ds(ci * rows, rows), :],
                    x_vmem.at[bi, pl.ds(ci * rows, rows), :],
                    copy_sems.at[bi * NCHUNK + ci],
                )
                cp.start()
                copies.append(cp)
        for bi in range(b):
            s = None
            sq = None
            for ci in range(NCHUNK):
                copies[bi * NCHUNK + ci].wait()
                xb = x_vmem[bi, pl.ds(ci * rows, rows), :]
                ps = jnp.sum(xb, axis=0, keepdims=True)
                psq = jnp.sum(xb * xb, axis=0, keepdims=True)
                s = ps if s is None else s + ps
                sq = psq if sq is None else sq + psq
            stats_ref[N_DEV - 1, 2 * bi : 2 * bi + 1, :] = s
            stats_ref[N_DEV - 1, 2 * bi + 1 : 2 * bi + 2, :] = sq

        barrier_sem = pltpu.get_barrier_semaphore()
        for d in (1, 2, 3):
            pl.semaphore_signal(
                barrier_sem,
                inc=1,
                device_id=((my + d) % N_DEV,),
                device_id_type=pl.DeviceIdType.MESH,
            )
        pl.semaphore_wait(barrier_sem, N_DEV - 1)

        sends = []
        for d in (1, 2, 3):
            rdma = pltpu.make_async_remote_copy(
                src_ref=stats_ref.at[N_DEV - 1],
                dst_ref=stats_ref.at[d - 1],
                send_sem=send_sems.at[d - 1],
                recv_sem=recv_sems.at[d - 1],
                device_id=((my + d) % N_DEV,),
                device_id_type=pl.DeviceIdType.MESH,
            )
            rdma.start()
            sends.append(rdma)
        for d in (1, 2, 3):
            recv = pltpu.make_async_remote_copy(
                src_ref=stats_ref.at[N_DEV - 1],
                dst_ref=stats_ref.at[d - 1],
                send_sem=send_sems.at[d - 1],
                recv_sem=recv_sems.at[d - 1],
                device_id=((my + d) % N_DEV,),
                device_id_type=pl.DeviceIdType.MESH,
            )
            recv.wait_recv()
        for rdma in sends:
            rdma.wait_send()

        eps = 1e-5
        inv_n = 1.0 / float(n_global)
        means = []
        scales = []
        for bi in range(b):
            ssum = (
                stats_ref[0, 2 * bi : 2 * bi + 1, :]
                + stats_ref[1, 2 * bi : 2 * bi + 1, :]
                + stats_ref[2, 2 * bi : 2 * bi + 1, :]
                + stats_ref[3, 2 * bi : 2 * bi + 1, :]
            )
            ssq = (
                stats_ref[0, 2 * bi + 1 : 2 * bi + 2, :]
                + stats_ref[1, 2 * bi + 1 : 2 * bi + 2, :]
                + stats_ref[2, 2 * bi + 1 : 2 * bi + 2, :]
                + stats_ref[3, 2 * bi + 1 : 2 * bi + 2, :]
            )
            mean = ssum * inv_n
            var = ssq * inv_n - mean * mean
            means.append(mean)
            scales.append(lax.rsqrt(var + eps))

        wp = wp_ref[:, :]
        out_waits = [None, None]
        k = 0
        for bi in range(b):
            for ci in range(NCHUNK):
                slot = k % 2
                if out_waits[slot] is not None:
                    out_waits[slot].wait()
                xb = x_vmem[bi, pl.ds(ci * rows, rows), :]
                hh = (xb - means[bi]) * scales[bi]
                a = hh * jax.nn.sigmoid(hh)
                outbuf[slot] = jnp.dot(
                    a, wp, preferred_element_type=jnp.float32
                )
                cp = pltpu.make_async_copy(
                    outbuf.at[slot],
                    out_hbm.at[bi, pl.ds(ci * rows, rows), :],
                    out_sems.at[slot],
                )
                cp.start()
                out_waits[slot] = cp
                k += 1
        out_waits[0].wait()
        out_waits[1].wait()

    out = pl.pallas_call(
        body,
        out_shape=jax.ShapeDtypeStruct((b, hw, cout), jnp.float32),
        in_specs=[
            pl.BlockSpec(memory_space=pltpu.MemorySpace.HBM),
            pl.BlockSpec(memory_space=pltpu.MemorySpace.VMEM),
        ],
        out_specs=pl.BlockSpec(memory_space=pltpu.MemorySpace.HBM),
        scratch_shapes=[
            pltpu.VMEM((b, hw, c), jnp.float32),
            pltpu.VMEM((2, rows, cout), jnp.float32),
            pltpu.VMEM((N_DEV, 2 * b, c), jnp.float32),
            pltpu.SemaphoreType.DMA((b * NCHUNK,)),
            pltpu.SemaphoreType.DMA((2,)),
            pltpu.SemaphoreType.DMA((N_DEV - 1,)),
            pltpu.SemaphoreType.DMA((N_DEV - 1,)),
        ],
        compiler_params=pltpu.CompilerParams(collective_id=0),
    )(x3, Wp)
    return out.reshape(b, h, w, cout)
